# baseline (speedup 1.0000x reference)
import functools

import jax
import jax.numpy as jnp
from jax import lax
from jax.experimental import pallas as pl
from jax.experimental.pallas import tpu as pltpu
from jax.experimental.pallas import tpu_sc as plsc

N = 50000
E = 800000
F = 64
CH = 16
NS = 16
DUMMY = N
OUT_ROWS = 51200
SLAB = OUT_ROWS // NS
E_PAD = 819200
R = E_PAD // 128
RT = R // NS
RD = R // (2 * NS)

NP = OUT_ROWS
BR = 2048
BRR = BR * CH // 128
MR = OUT_ROWS * CH // 128
G = NP // BR
EPS = 1e-5
SLOPE = 0.01


def _leaky(x):
    return jnp.where(x >= 0, x, SLOPE * x)


def _sc_agg_body(m_hbm, srcs_hbm, dst_hbm, zrow_hbm, out_hbm,
                 idxbig, dstbig, rowsA, rowsB,
                 gsem, ssemA, ssemB, acc_sh):
    c = lax.axis_index("c")
    s = lax.axis_index("s")
    base = s * RT
    for q in range(2):
        chunk = 2 * c + q
        pltpu.sync_copy(zrow_hbm, acc_sh.at[pl.ds(s * SLAB, SLAB)])
        plsc.subcore_barrier()

        def super_body(u, carry):
            @pl.when(u > 0)
            def _():
                for j in range(4):
                    pltpu.make_async_copy(rowsB.at[pl.ds(j * 128, 128)],
                                          acc_sh.at[idxbig.at[4 + j]],
                                          ssemB).wait()

            row0 = base + u * 80
            pltpu.sync_copy(srcs_hbm.at[chunk, pl.ds(row0, 80)], idxbig)
            pltpu.sync_copy(dst_hbm.at[pl.ds(row0, 80)], dstbig)

            def body(t, carry2):
                r8 = t * 8
                gA = [pltpu.async_copy(m_hbm.at[idxbig.at[r8 + j]],
                                       rowsA.at[pl.ds(j * 128, 128)], gsem)
                      for j in range(4)]

                @pl.when(t > 0)
                def _():
                    for j in range(4):
                        pltpu.make_async_copy(rowsB.at[pl.ds(j * 128, 128)],
                                              acc_sh.at[dstbig.at[4 + j]],
                                              ssemB).wait()

                gB = [pltpu.async_copy(m_hbm.at[idxbig.at[r8 + 4 + j]],
                                       rowsB.at[pl.ds(j * 128, 128)], gsem)
                      for j in range(4)]
                for cp in gA:
                    cp.wait()
                sA = [pltpu.async_copy(rowsA.at[pl.ds(j * 128, 128)],
                                       acc_sh.at[dstbig.at[r8 + j]],
                                       ssemA, add=True)
                      for j in range(4)]
                for cp in gB:
                    cp.wait()
                for j in range(4):
                    pltpu.async_copy(rowsB.at[pl.ds(j * 128, 128)],
                                     acc_sh.at[dstbig.at[r8 + 4 + j]],
                                     ssemB, add=True)
                for cp in sA:
                    cp.wait()
                return carry2

            lax.fori_loop(0, 10, body, 0)
            return carry

        lax.fori_loop(0, RT // 80, super_body, 0)
        for j in range(4):
            pltpu.make_async_copy(rowsB.at[pl.ds(j * 128, 128)],
                                  acc_sh.at[idxbig.at[4 + j]], ssemB).wait()
        plsc.subcore_barrier()
        o = s * SLAB
        pltpu.sync_copy(acc_sh.at[pl.ds(o, SLAB)],
                        out_hbm.at[chunk, pl.ds(o, SLAB)])


@functools.cache
def _sc_agg():
    mesh = plsc.VectorSubcoreMesh(core_axis_name="c", subcore_axis_name="s")
    return pl.kernel(
        _sc_agg_body,
        out_type=jax.ShapeDtypeStruct((4, OUT_ROWS, CH), jnp.float32),
        mesh=mesh,
        compiler_params=pltpu.CompilerParams(use_tc_tiling_on_sc=False),
        scratch_types=[
            pltpu.VMEM((80, 128), jnp.int32),
            pltpu.VMEM((80, 128), jnp.int32),
            pltpu.VMEM((512, CH), jnp.float32),
            pltpu.VMEM((512, CH), jnp.float32),
            pltpu.SemaphoreType.DMA,
            pltpu.SemaphoreType.DMA,
            pltpu.SemaphoreType.DMA,
            pltpu.VMEM_SHARED((OUT_ROWS, CH), jnp.float32),
        ],
    )


def _sc_deg_body(dst_hbm, z16_hbm, ones_hbm, out_hbm,
                 ones_v, dsti_v, sem, acc_sh):
    c = lax.axis_index("c")
    s = lax.axis_index("s")
    pltpu.sync_copy(z16_hbm, acc_sh.at[pl.ds(s * SLAB, SLAB)])
    pltpu.sync_copy(ones_hbm, ones_v)
    plsc.subcore_barrier()
    base = (c * NS + s) * RD

    def body(k, carry):
        row0 = base + k * 8
        pltpu.sync_copy(dst_hbm.at[pl.ds(row0, 8)], dsti_v)
        scs = [pltpu.async_copy(ones_v, acc_sh.at[dsti_v.at[j]], sem, add=True)
               for j in range(8)]
        for cp in scs:
            cp.wait()
        return carry

    lax.fori_loop(0, RD // 8, body, 0)
    plsc.subcore_barrier()
    o = s * SLAB
    pltpu.sync_copy(acc_sh.at[pl.ds(o, SLAB)], out_hbm.at[c, pl.ds(o, SLAB)])


@functools.cache
def _sc_deg():
    mesh = plsc.VectorSubcoreMesh(core_axis_name="c", subcore_axis_name="s")
    return pl.kernel(
        _sc_deg_body,
        out_type=jax.ShapeDtypeStruct((2, OUT_ROWS, 16), jnp.float32),
        mesh=mesh,
        compiler_params=pltpu.CompilerParams(use_tc_tiling_on_sc=False),
        scratch_types=[
            pltpu.VMEM((128, 16), jnp.float32),
            pltpu.VMEM((8, 128), jnp.int32),
            pltpu.SemaphoreType.DMA,
            pltpu.VMEM_SHARED((OUT_ROWS, 16), jnp.float32),
        ],
    )


def _row_mask(i):
    r = lax.broadcasted_iota(jnp.int32, (BR, 1), 0) + i * BR
    return r < N


def _enc1_body(x_ref, w_ref, b_ref, z_ref, st_ref, acc):
    i = pl.program_id(0)
    z = jnp.dot(x_ref[...], w_ref[...], preferred_element_type=jnp.float32)
    z = z + b_ref[0:1, :]
    z_ref[...] = z

    @pl.when(i == 0)
    def _():
        acc[...] = jnp.zeros_like(acc)

    zm = jnp.where(_row_mask(i), z, 0.0)
    acc[0:1, :] += jnp.sum(zm, axis=0, keepdims=True)
    acc[1:2, :] += jnp.sum(zm * zm, axis=0, keepdims=True)

    @pl.when(i == G - 1)
    def _():
        st_ref[...] = acc[...]


def _enc2_body(z_ref, st_ref, g_ref, be_ref, w_ref, b_ref, z2_ref, st2_ref, acc):
    i = pl.program_id(0)
    mu = st_ref[0:1, :] / N
    var = st_ref[1:2, :] / N - mu * mu
    y = g_ref[0:1, :] * (z_ref[...] - mu) / jnp.sqrt(var + EPS) + be_ref[0:1, :]
    a = _leaky(y)
    z2 = jnp.dot(a, w_ref[...], preferred_element_type=jnp.float32)
    z2 = z2 + b_ref[0:1, :]
    z2_ref[...] = z2

    @pl.when(i == 0)
    def _():
        acc[...] = jnp.zeros_like(acc)

    zm = jnp.where(_row_mask(i), z2, 0.0)
    acc[0:1, :] += jnp.sum(zm, axis=0, keepdims=True)
    acc[1:2, :] += jnp.sum(zm * zm, axis=0, keepdims=True)

    @pl.when(i == G - 1)
    def _():
        st2_ref[...] = acc[...]


def _enc3_body(z_ref, st_ref, g_ref, be_ref, wn_ref, dp_ref,
               h_ref, dinv_ref, m_ref):
    mu = st_ref[0:1, :] / N
    var = st_ref[1:2, :] / N - mu * mu
    y = g_ref[0:1, :] * (z_ref[...] - mu) / jnp.sqrt(var + EPS) + be_ref[0:1, :]
    h = _leaky(y)
    h_ref[...] = h
    d = dp_ref[0] + dp_ref[1]
    dinv_ref[...] = 1.0 / jnp.maximum(d, 1.0)
    mm = jnp.dot(h, wn_ref[...], preferred_element_type=jnp.float32)
    for q in range(4):
        m_ref[q] = mm[:, q * CH:(q + 1) * CH]


def _combine_body(h_ref, s_ref, dinv_ref, ws_ref, b_ref, wn_ref,
                  h2_ref, m_ref):
    sc = jnp.concatenate([s_ref[q] for q in range(4)],
                         axis=1) * dinv_ref[:, 0:1]
    hs = jnp.dot(h_ref[...], ws_ref[...], preferred_element_type=jnp.float32)
    h2 = _leaky(hs + sc + b_ref[0:1, :])
    h2_ref[...] = h2
    mm = jnp.dot(h2, wn_ref[...], preferred_element_type=jnp.float32)
    for q in range(4):
        m_ref[q] = mm[:, q * CH:(q + 1) * CH]


def _final_body(h_ref, s_ref, dinv_ref, ws_ref, b_ref, fw_ref, fb_ref, o_ref):
    sc = jnp.concatenate([s_ref[q] for q in range(4)],
                         axis=1) * dinv_ref[:, 0:1]
    hs = jnp.dot(h_ref[...], ws_ref[...], preferred_element_type=jnp.float32)
    h2 = _leaky(hs + sc + b_ref[0:1, :])
    o = jnp.dot(h2, fw_ref[...], preferred_element_type=jnp.float32)
    o_ref[...] = o + fb_ref[0:1, :]


def _row_spec(w):
    return pl.BlockSpec((BR, w), lambda i: (i, 0))


def _full_spec(r, c):
    return pl.BlockSpec((r, c), lambda i: (0, 0))


_SPLIT_SPEC = pl.BlockSpec((4, BR, CH), lambda i: (0, i, 0))

_enc1 = pl.pallas_call(
    _enc1_body,
    grid=(G,),
    in_specs=[_row_spec(F), _full_spec(F, F), _full_spec(8, F)],
    out_specs=[_row_spec(F), _full_spec(8, F)],
    out_shape=[jax.ShapeDtypeStruct((NP, F), jnp.float32),
               jax.ShapeDtypeStruct((8, F), jnp.float32)],
    scratch_shapes=[pltpu.VMEM((8, F), jnp.float32)],
)

_enc2 = pl.pallas_call(
    _enc2_body,
    grid=(G,),
    in_specs=[_row_spec(F), _full_spec(8, F), _full_spec(8, F),
              _full_spec(8, F), _full_spec(F, F), _full_spec(8, F)],
    out_specs=[_row_spec(F), _full_spec(8, F)],
    out_shape=[jax.ShapeDtypeStruct((NP, F), jnp.float32),
               jax.ShapeDtypeStruct((8, F), jnp.float32)],
    scratch_shapes=[pltpu.VMEM((8, F), jnp.float32)],
)

_enc3 = pl.pallas_call(
    _enc3_body,
    grid=(G,),
    in_specs=[_row_spec(F), _full_spec(8, F), _full_spec(8, F),
              _full_spec(8, F), _full_spec(F, F),
              pl.BlockSpec((2, BR, 16), lambda i: (0, i, 0))],
    out_specs=[_row_spec(F), _row_spec(16), _SPLIT_SPEC],
    out_shape=[jax.ShapeDtypeStruct((NP, F), jnp.float32),
               jax.ShapeDtypeStruct((NP, 16), jnp.float32),
               jax.ShapeDtypeStruct((4, NP, CH), jnp.float32)],
)

_combine = pl.pallas_call(
    _combine_body,
    grid=(G,),
    in_specs=[_row_spec(F), _SPLIT_SPEC, _row_spec(16),
              _full_spec(F, F), _full_spec(8, F), _full_spec(F, F)],
    out_specs=[_row_spec(F), _SPLIT_SPEC],
    out_shape=[jax.ShapeDtypeStruct((NP, F), jnp.float32),
               jax.ShapeDtypeStruct((4, NP, CH), jnp.float32)],
)

_final = pl.pallas_call(
    _final_body,
    grid=(G,),
    in_specs=[_row_spec(F), _SPLIT_SPEC, _row_spec(16),
              _full_spec(F, F), _full_spec(8, F), _full_spec(F, F),
              _full_spec(8, F)],
    out_specs=_row_spec(F),
    out_shape=jax.ShapeDtypeStruct((NP, F), jnp.float32),
)


def kernel(x, edge_index, enc_params, conv_params, fc_W, fc_b):
    (W0, b0, g0, be0), (W1, b1, g1, be1) = enc_params
    f_in = x.shape[1]

    xp = jnp.pad(x, ((0, NP - N), (0, F - f_in)))
    W0p = jnp.pad(W0, ((0, F - f_in), (0, 0)))
    fcWp = jnp.pad(fc_W, ((0, 0), (0, F - fc_W.shape[1])))
    fcbp = jnp.pad(fc_b, (0, F - fc_b.shape[0]))

    def row8(v):
        return jnp.broadcast_to(v[None, :], (8, F))

    src = edge_index[0]
    dst = edge_index[1]
    pad = E_PAD - E
    src_p = jnp.concatenate([src, jnp.zeros((pad,), jnp.int32)])
    dst_p = jnp.concatenate([dst, jnp.full((pad,), DUMMY, jnp.int32)])
    srcs = jnp.stack([src_p + j * OUT_ROWS for j in range(4)]).reshape(4, R, 128)
    dstT = dst_p.reshape(R, 128)
    z32 = jnp.zeros((SLAB, CH), jnp.float32)
    z16 = jnp.zeros((SLAB, 16), jnp.float32)
    ones16 = jnp.ones((128, 16), jnp.float32)

    dp = _sc_deg()(dstT, z16, ones16)

    z1, st1 = _enc1(xp, W0p, row8(b0))
    z2, st2 = _enc2(z1, st1, row8(g0), row8(be0), W1, row8(b1))
    h, dinv, m = _enc3(z2, st2, row8(g1), row8(be1), conv_params[0][1], dp)

    for t in range(len(conv_params)):
        Ws, Wn, bb = conv_params[t]
        s = _sc_agg()(m.reshape(4 * OUT_ROWS, CH), srcs, dstT, z32)
        if t + 1 < len(conv_params):
            h, m = _combine(h, s, dinv, Ws, row8(bb),
                            conv_params[t + 1][1])
        else:
            o = _final(h, s, dinv, Ws, row8(bb), fcWp, row8(fcbp))
    return o[:N, :fc_b.shape[0]]


# speedup vs baseline: 1.0096x; 1.0096x over previous
import functools

import jax
import jax.numpy as jnp
from jax import lax
from jax.experimental import pallas as pl
from jax.experimental.pallas import tpu as pltpu
from jax.experimental.pallas import tpu_sc as plsc

N = 50000
E = 800000
F = 64
CH = 16
NS = 16
DUMMY = N
OUT_ROWS = 50176
SLAB = OUT_ROWS // NS
E_PAD = 819200
R = E_PAD // 128
RT = R // NS
RD = R // (2 * NS)

NP = OUT_ROWS
BR = 1024
BRR = BR * CH // 128
MR = OUT_ROWS * CH // 128
G = NP // BR
EPS = 1e-5
SLOPE = 0.01


def _leaky(x):
    return jnp.where(x >= 0, x, SLOPE * x)


def _sc_agg_body(m_hbm, srcs_hbm, dst_hbm, zrow_hbm, out_hbm,
                 idxbig, dstbig, rowsA, rowsB,
                 gsem, ssemA, ssemB, acc_sh):
    c = lax.axis_index("c")
    s = lax.axis_index("s")
    base = s * RT
    for q in range(2):
        chunk = 2 * c + q
        pltpu.sync_copy(zrow_hbm, acc_sh.at[pl.ds(s * SLAB, SLAB)])
        plsc.subcore_barrier()

        def super_body(u, carry):
            @pl.when(u > 0)
            def _():
                for j in range(4):
                    pltpu.make_async_copy(rowsB.at[pl.ds(j * 128, 128)],
                                          acc_sh.at[idxbig.at[4 + j]],
                                          ssemB).wait()

            row0 = base + u * 80
            pltpu.sync_copy(srcs_hbm.at[chunk, pl.ds(row0, 80)], idxbig)
            pltpu.sync_copy(dst_hbm.at[pl.ds(row0, 80)], dstbig)

            def body(t, carry2):
                r8 = t * 8
                gA = [pltpu.async_copy(m_hbm.at[idxbig.at[r8 + j]],
                                       rowsA.at[pl.ds(j * 128, 128)], gsem)
                      for j in range(4)]

                @pl.when(t > 0)
                def _():
                    for j in range(4):
                        pltpu.make_async_copy(rowsB.at[pl.ds(j * 128, 128)],
                                              acc_sh.at[dstbig.at[4 + j]],
                                              ssemB).wait()

                gB = [pltpu.async_copy(m_hbm.at[idxbig.at[r8 + 4 + j]],
                                       rowsB.at[pl.ds(j * 128, 128)], gsem)
                      for j in range(4)]
                for cp in gA:
                    cp.wait()
                sA = [pltpu.async_copy(rowsA.at[pl.ds(j * 128, 128)],
                                       acc_sh.at[dstbig.at[r8 + j]],
                                       ssemA, add=True)
                      for j in range(4)]
                for cp in gB:
                    cp.wait()
                for j in range(4):
                    pltpu.async_copy(rowsB.at[pl.ds(j * 128, 128)],
                                     acc_sh.at[dstbig.at[r8 + 4 + j]],
                                     ssemB, add=True)
                for cp in sA:
                    cp.wait()
                return carry2

            lax.fori_loop(0, 10, body, 0)
            return carry

        lax.fori_loop(0, RT // 80, super_body, 0)
        for j in range(4):
            pltpu.make_async_copy(rowsB.at[pl.ds(j * 128, 128)],
                                  acc_sh.at[idxbig.at[4 + j]], ssemB).wait()
        plsc.subcore_barrier()
        o = s * SLAB
        pltpu.sync_copy(acc_sh.at[pl.ds(o, SLAB)],
                        out_hbm.at[chunk, pl.ds(o, SLAB)])


@functools.cache
def _sc_agg():
    mesh = plsc.VectorSubcoreMesh(core_axis_name="c", subcore_axis_name="s")
    return pl.kernel(
        _sc_agg_body,
        out_type=jax.ShapeDtypeStruct((4, OUT_ROWS, CH), jnp.float32),
        mesh=mesh,
        compiler_params=pltpu.CompilerParams(use_tc_tiling_on_sc=False),
        scratch_types=[
            pltpu.VMEM((80, 128), jnp.int32),
            pltpu.VMEM((80, 128), jnp.int32),
            pltpu.VMEM((512, CH), jnp.float32),
            pltpu.VMEM((512, CH), jnp.float32),
            pltpu.SemaphoreType.DMA,
            pltpu.SemaphoreType.DMA,
            pltpu.SemaphoreType.DMA,
            pltpu.VMEM_SHARED((OUT_ROWS, CH), jnp.float32),
        ],
    )


def _sc_deg_body(dst_hbm, z16_hbm, ones_hbm, out_hbm,
                 ones_v, dsti_v, sem, acc_sh):
    c = lax.axis_index("c")
    s = lax.axis_index("s")
    pltpu.sync_copy(z16_hbm, acc_sh.at[pl.ds(s * SLAB, SLAB)])
    pltpu.sync_copy(ones_hbm, ones_v)
    plsc.subcore_barrier()
    base = (c * NS + s) * RD

    def body(k, carry):
        row0 = base + k * 8
        pltpu.sync_copy(dst_hbm.at[pl.ds(row0, 8)], dsti_v)
        scs = [pltpu.async_copy(ones_v, acc_sh.at[dsti_v.at[j]], sem, add=True)
               for j in range(8)]
        for cp in scs:
            cp.wait()
        return carry

    lax.fori_loop(0, RD // 8, body, 0)
    plsc.subcore_barrier()
    o = s * SLAB
    pltpu.sync_copy(acc_sh.at[pl.ds(o, SLAB)], out_hbm.at[c, pl.ds(o, SLAB)])


@functools.cache
def _sc_deg():
    mesh = plsc.VectorSubcoreMesh(core_axis_name="c", subcore_axis_name="s")
    return pl.kernel(
        _sc_deg_body,
        out_type=jax.ShapeDtypeStruct((2, OUT_ROWS, 16), jnp.float32),
        mesh=mesh,
        compiler_params=pltpu.CompilerParams(use_tc_tiling_on_sc=False),
        scratch_types=[
            pltpu.VMEM((128, 16), jnp.float32),
            pltpu.VMEM((8, 128), jnp.int32),
            pltpu.SemaphoreType.DMA,
            pltpu.VMEM_SHARED((OUT_ROWS, 16), jnp.float32),
        ],
    )


def _row_mask(i):
    r = lax.broadcasted_iota(jnp.int32, (BR, 1), 0) + i * BR
    return r < N


def _enc1_body(x_ref, w_ref, b_ref, z_ref, st_ref, acc):
    i = pl.program_id(0)
    z = jnp.dot(x_ref[...], w_ref[...], preferred_element_type=jnp.float32)
    z = z + b_ref[0:1, :]
    z_ref[...] = z

    @pl.when(i == 0)
    def _():
        acc[...] = jnp.zeros_like(acc)

    zm = jnp.where(_row_mask(i), z, 0.0)
    acc[0:1, :] += jnp.sum(zm, axis=0, keepdims=True)
    acc[1:2, :] += jnp.sum(zm * zm, axis=0, keepdims=True)

    @pl.when(i == G - 1)
    def _():
        st_ref[...] = acc[...]


def _enc2_body(z_ref, st_ref, g_ref, be_ref, w_ref, b_ref, z2_ref, st2_ref, acc):
    i = pl.program_id(0)
    mu = st_ref[0:1, :] / N
    var = st_ref[1:2, :] / N - mu * mu
    y = g_ref[0:1, :] * (z_ref[...] - mu) / jnp.sqrt(var + EPS) + be_ref[0:1, :]
    a = _leaky(y)
    z2 = jnp.dot(a, w_ref[...], preferred_element_type=jnp.float32)
    z2 = z2 + b_ref[0:1, :]
    z2_ref[...] = z2

    @pl.when(i == 0)
    def _():
        acc[...] = jnp.zeros_like(acc)

    zm = jnp.where(_row_mask(i), z2, 0.0)
    acc[0:1, :] += jnp.sum(zm, axis=0, keepdims=True)
    acc[1:2, :] += jnp.sum(zm * zm, axis=0, keepdims=True)

    @pl.when(i == G - 1)
    def _():
        st2_ref[...] = acc[...]


def _enc3_body(z_ref, st_ref, g_ref, be_ref, wn_ref, dp_ref,
               h_ref, dinv_ref, m_ref):
    mu = st_ref[0:1, :] / N
    var = st_ref[1:2, :] / N - mu * mu
    y = g_ref[0:1, :] * (z_ref[...] - mu) / jnp.sqrt(var + EPS) + be_ref[0:1, :]
    h = _leaky(y)
    h_ref[...] = h
    d = dp_ref[0] + dp_ref[1]
    dinv_ref[...] = 1.0 / jnp.maximum(d, 1.0)
    mm = jnp.dot(h, wn_ref[...], preferred_element_type=jnp.float32)
    for q in range(4):
        m_ref[q] = mm[:, q * CH:(q + 1) * CH]


def _combine_body(h_ref, s_ref, dinv_ref, ws_ref, b_ref, wn_ref,
                  h2_ref, m_ref):
    sc = jnp.concatenate([s_ref[q] for q in range(4)],
                         axis=1) * dinv_ref[:, 0:1]
    hs = jnp.dot(h_ref[...], ws_ref[...], preferred_element_type=jnp.float32)
    h2 = _leaky(hs + sc + b_ref[0:1, :])
    h2_ref[...] = h2
    mm = jnp.dot(h2, wn_ref[...], preferred_element_type=jnp.float32)
    for q in range(4):
        m_ref[q] = mm[:, q * CH:(q + 1) * CH]


def _final_body(h_ref, s_ref, dinv_ref, ws_ref, b_ref, fw_ref, fb_ref, o_ref):
    sc = jnp.concatenate([s_ref[q] for q in range(4)],
                         axis=1) * dinv_ref[:, 0:1]
    hs = jnp.dot(h_ref[...], ws_ref[...], preferred_element_type=jnp.float32)
    h2 = _leaky(hs + sc + b_ref[0:1, :])
    o = jnp.dot(h2, fw_ref[...], preferred_element_type=jnp.float32)
    o_ref[...] = o + fb_ref[0:1, :]


def _row_spec(w):
    return pl.BlockSpec((BR, w), lambda i: (i, 0))


def _full_spec(r, c):
    return pl.BlockSpec((r, c), lambda i: (0, 0))


_SPLIT_SPEC = pl.BlockSpec((4, BR, CH), lambda i: (0, i, 0))

_enc1 = pl.pallas_call(
    _enc1_body,
    grid=(G,),
    in_specs=[_row_spec(F), _full_spec(F, F), _full_spec(8, F)],
    out_specs=[_row_spec(F), _full_spec(8, F)],
    out_shape=[jax.ShapeDtypeStruct((NP, F), jnp.float32),
               jax.ShapeDtypeStruct((8, F), jnp.float32)],
    scratch_shapes=[pltpu.VMEM((8, F), jnp.float32)],
)

_enc2 = pl.pallas_call(
    _enc2_body,
    grid=(G,),
    in_specs=[_row_spec(F), _full_spec(8, F), _full_spec(8, F),
              _full_spec(8, F), _full_spec(F, F), _full_spec(8, F)],
    out_specs=[_row_spec(F), _full_spec(8, F)],
    out_shape=[jax.ShapeDtypeStruct((NP, F), jnp.float32),
               jax.ShapeDtypeStruct((8, F), jnp.float32)],
    scratch_shapes=[pltpu.VMEM((8, F), jnp.float32)],
)

_enc3 = pl.pallas_call(
    _enc3_body,
    grid=(G,),
    in_specs=[_row_spec(F), _full_spec(8, F), _full_spec(8, F),
              _full_spec(8, F), _full_spec(F, F),
              pl.BlockSpec((2, BR, 16), lambda i: (0, i, 0))],
    out_specs=[_row_spec(F), _row_spec(16), _SPLIT_SPEC],
    out_shape=[jax.ShapeDtypeStruct((NP, F), jnp.float32),
               jax.ShapeDtypeStruct((NP, 16), jnp.float32),
               jax.ShapeDtypeStruct((4, NP, CH), jnp.float32)],
)

_combine = pl.pallas_call(
    _combine_body,
    grid=(G,),
    in_specs=[_row_spec(F), _SPLIT_SPEC, _row_spec(16),
              _full_spec(F, F), _full_spec(8, F), _full_spec(F, F)],
    out_specs=[_row_spec(F), _SPLIT_SPEC],
    out_shape=[jax.ShapeDtypeStruct((NP, F), jnp.float32),
               jax.ShapeDtypeStruct((4, NP, CH), jnp.float32)],
)

_final = pl.pallas_call(
    _final_body,
    grid=(G,),
    in_specs=[_row_spec(F), _SPLIT_SPEC, _row_spec(16),
              _full_spec(F, F), _full_spec(8, F), _full_spec(F, F),
              _full_spec(8, F)],
    out_specs=_row_spec(F),
    out_shape=jax.ShapeDtypeStruct((NP, F), jnp.float32),
)


def kernel(x, edge_index, enc_params, conv_params, fc_W, fc_b):
    (W0, b0, g0, be0), (W1, b1, g1, be1) = enc_params
    f_in = x.shape[1]

    xp = jnp.pad(x, ((0, NP - N), (0, F - f_in)))
    W0p = jnp.pad(W0, ((0, F - f_in), (0, 0)))
    fcWp = jnp.pad(fc_W, ((0, 0), (0, F - fc_W.shape[1])))
    fcbp = jnp.pad(fc_b, (0, F - fc_b.shape[0]))

    def row8(v):
        return jnp.broadcast_to(v[None, :], (8, F))

    src = edge_index[0]
    dst = edge_index[1]
    pad = E_PAD - E
    src_p = jnp.concatenate([src, jnp.zeros((pad,), jnp.int32)])
    dst_p = jnp.concatenate([dst, jnp.full((pad,), DUMMY, jnp.int32)])
    srcs = jnp.stack([src_p + j * OUT_ROWS for j in range(4)]).reshape(4, R, 128)
    dstT = dst_p.reshape(R, 128)
    z32 = jnp.zeros((SLAB, CH), jnp.float32)
    z16 = jnp.zeros((SLAB, 16), jnp.float32)
    ones16 = jnp.ones((128, 16), jnp.float32)

    dp = _sc_deg()(dstT, z16, ones16)

    z1, st1 = _enc1(xp, W0p, row8(b0))
    z2, st2 = _enc2(z1, st1, row8(g0), row8(be0), W1, row8(b1))
    h, dinv, m = _enc3(z2, st2, row8(g1), row8(be1), conv_params[0][1], dp)

    for t in range(len(conv_params)):
        Ws, Wn, bb = conv_params[t]
        s = _sc_agg()(m.reshape(4 * OUT_ROWS, CH), srcs, dstT, z32)
        if t + 1 < len(conv_params):
            h, m = _combine(h, s, dinv, Ws, row8(bb),
                            conv_params[t + 1][1])
        else:
            o = _final(h, s, dinv, Ws, row8(bb), fcWp, row8(fcbp))
    return o[:N, :fc_b.shape[0]]
